# baseline (speedup 1.0000x reference)
import jax
import jax.numpy as jnp
from jax import lax
from jax.experimental import pallas as pl
from jax.experimental.pallas import tpu as pltpu


def _pos_body(t_ref, o_ref):
    r, bs, d = o_ref.shape
    base = pl.program_id(0) * r
    rids = base + lax.broadcasted_iota(jnp.int32, (r, 32), 0)
    cids = lax.broadcasted_iota(jnp.int32, (r, 32), 1)
    sel = (cids == rids % 10)
    sel |= (cids == 10 + (rids // 10) % 10)
    sel |= (cids == 20 + rids // 100)
    s = sel.astype(jnp.float32)
    pos = jax.lax.dot_general(
        s, t_ref[...],
        dimension_numbers=(((1,), (0,)), ((), ())),
        preferred_element_type=jnp.float32,
        precision=jax.lax.Precision.HIGHEST,
    )
    o_ref[...] = jnp.broadcast_to(pos[:, None, :], (r, bs, d))


def kernel(x, row_embed, col_embed, time_embed):
    bs, frame_num, h, w = x.shape[:4]
    d4 = row_embed.shape[1]
    d2 = time_embed.shape[1]
    d = 2 * d4 + d2
    n = frame_num * h * w

    t = jnp.zeros((32, d), jnp.float32)
    t = t.at[0:10, 0:d4].set(row_embed)
    t = t.at[10:20, d4:2 * d4].set(col_embed)
    t = t.at[20:30, 2 * d4:d].set(time_embed)

    r = 250
    out = pl.pallas_call(
        _pos_body,
        grid=(n // r,),
        in_specs=[pl.BlockSpec((32, d), lambda i: (0, 0))],
        out_specs=pl.BlockSpec((r, bs, d), lambda i: (i, 0, 0)),
        out_shape=jax.ShapeDtypeStruct((n, bs, d), jnp.float32),
    )(t)
    out = out.reshape(frame_num, h, w, bs, d)
    return jnp.transpose(out, (3, 0, 1, 2, 4))


# speedup vs baseline: 1.0610x; 1.0610x over previous
import jax
import jax.numpy as jnp
from jax import lax
from jax.experimental import pallas as pl
from jax.experimental.pallas import tpu as pltpu


def _pos_body(t_ref, o_ref):
    r, bs, d = o_ref.shape
    base = pl.program_id(0) * r
    rids = base + lax.broadcasted_iota(jnp.int32, (r, 32), 0)
    cids = lax.broadcasted_iota(jnp.int32, (r, 32), 1)
    sel = (cids == rids % 10)
    sel |= (cids == 10 + (rids // 10) % 10)
    sel |= (cids == 20 + rids // 100)
    s = sel.astype(jnp.float32)
    pos = jax.lax.dot_general(
        s, t_ref[...],
        dimension_numbers=(((1,), (0,)), ((), ())),
        preferred_element_type=jnp.float32,
        precision=jax.lax.Precision.HIGHEST,
    )
    o_ref[...] = jnp.broadcast_to(pos[:, None, :], (r, bs, d))


def kernel(x, row_embed, col_embed, time_embed):
    bs, frame_num, h, w = x.shape[:4]
    d4 = row_embed.shape[1]
    d2 = time_embed.shape[1]
    d = 2 * d4 + d2
    n = frame_num * h * w

    t = jnp.zeros((32, d), jnp.float32)
    t = t.at[0:10, 0:d4].set(row_embed)
    t = t.at[10:20, d4:2 * d4].set(col_embed)
    t = t.at[20:30, 2 * d4:d].set(time_embed)

    r = 100
    out = pl.pallas_call(
        _pos_body,
        grid=(n // r,),
        in_specs=[pl.BlockSpec((32, d), lambda i: (0, 0))],
        out_specs=pl.BlockSpec((r, bs, d), lambda i: (i, 0, 0)),
        out_shape=jax.ShapeDtypeStruct((n, bs, d), jnp.float32),
    )(t)
    out = out.reshape(frame_num, h, w, bs, d)
    return jnp.transpose(out, (3, 0, 1, 2, 4))
